# baseline (speedup 1.0000x reference)
import jax
import jax.numpy as jnp
from jax import lax
from jax.experimental import pallas as pl
from jax.experimental.pallas import tpu as pltpu
from jax.experimental.pallas import tpu_sc as plsc

B = 16384
D = 64
L = 16
NC = 2
NS = 16
NW = NC * NS
ROWS = B // NW
CROWS = B // NC
CH = 128
NCH = ROWS // CH
NG = ROWS // L


def _sc_body(z_hbm, idx_hbm, bf_hbm, emb1_hbm, emb2_hbm,
             out_hbm, idx_v, a_v, b_v, z_v, bf_v, out_v, z_sh, sem, sem_sh):
    cid = lax.axis_index("c")
    sid = lax.axis_index("s")
    wid = cid * NS + sid
    base = wid * ROWS
    cbase = cid * CROWS

    @pl.when(sid == 0)
    def _start_big():
        pltpu.make_async_copy(z_hbm.at[pl.ds(cbase, CROWS)], z_sh,
                              sem_sh).start()

    copies = []

    def fire(src, dst):
        c = pltpu.make_async_copy(src, dst, sem)
        c.start()
        copies.append(c)

    pltpu.sync_copy(idx_hbm.at[pl.ds(wid * NCH, NCH)], idx_v)
    for j in range(NCH):
        fire(emb1_hbm.at[idx_v.at[j]], a_v.at[pl.ds(j * CH, CH)])
        fire(emb2_hbm.at[idx_v.at[j]], b_v.at[pl.ds(j * CH, CH)])
    fire(bf_hbm, bf_v)

    @pl.when(sid == 0)
    def _wait_big():
        pltpu.make_async_copy(z_hbm.at[pl.ds(cbase, CROWS)], z_sh,
                              sem_sh).wait()

    plsc.subcore_barrier()

    pltpu.sync_copy(z_sh.at[pl.ds(sid * ROWS, ROWS)], z_v)

    for c in copies:
        c.wait()

    bias_vec = bf_v[pl.ds(0, L)]
    lanes = lax.iota(jnp.int32, L)

    def group(g, carry):
        out16 = jnp.zeros((L,), jnp.float32)
        for rr in range(L):
            r = g * L + rr
            acc = z_v[r, pl.ds(0, L)] * a_v[r, pl.ds(0, L)]
            for k in range(1, D // L):
                acc = acc + z_v[r, pl.ds(k * L, L)] * a_v[r, pl.ds(k * L, L)]
            s = jnp.sum(acc)
            out16 = jnp.where(lanes == rr, s, out16)
        bv = b_v[pl.dslice(g * L, L)]
        out_v[pl.dslice(g * L, L)] = out16 + bv + bias_vec
        return carry

    lax.fori_loop(0, NG, group, 0)

    pltpu.sync_copy(out_v, out_hbm.at[pl.ds(base, ROWS)])


def _build_sc():
    mesh = plsc.VectorSubcoreMesh(core_axis_name="c", subcore_axis_name="s")
    return pl.kernel(
        _sc_body,
        out_type=jax.ShapeDtypeStruct((B,), jnp.float32),
        mesh=mesh,
        compiler_params=pltpu.CompilerParams(
            needs_layout_passes=False, use_tc_tiling_on_sc=False),
        scratch_types=[
            pltpu.VMEM((NCH, CH), jnp.int32),
            pltpu.VMEM((ROWS, D), jnp.float32),
            pltpu.VMEM((ROWS,), jnp.float32),
            pltpu.VMEM((ROWS, D), jnp.float32),
            pltpu.VMEM((L,), jnp.float32),
            pltpu.VMEM((ROWS,), jnp.float32),
            pltpu.VMEM_SHARED((CROWS, D), jnp.float32),
            pltpu.SemaphoreType.DMA,
            pltpu.SemaphoreType.DMA,
        ],
    )


_sc_kernel = _build_sc()

_TCB = 2048


def _tc_body(x_ref, w_ref, o_ref):
    o_ref[...] = jax.lax.dot_general(
        x_ref[...], w_ref[...], (((1,), (1,)), ((), ())),
        preferred_element_type=jnp.float32)


_tc_fixed = pl.pallas_call(
    _tc_body,
    out_shape=jax.ShapeDtypeStruct((B, 1), jnp.float32),
    grid=(B // _TCB,),
    in_specs=[
        pl.BlockSpec((_TCB, D), lambda i: (i, 0)),
        pl.BlockSpec((1, D), lambda i: (0, 0)),
    ],
    out_specs=pl.BlockSpec((_TCB, 1), lambda i: (i, 0)),
)


@jax.jit
def kernel(x, z, idx, W_f, b_f, emb1, emb2):
    bf16 = jnp.broadcast_to(b_f, (L,))
    idx2 = idx.astype(jnp.int32).reshape(NW * NCH, CH)
    rand = _sc_kernel(z, idx2, bf16, emb1, emb2.reshape(-1))
    fixed = _tc_fixed(x, W_f)
    return fixed + rand.reshape(B, 1)


# speedup vs baseline: 1.0053x; 1.0053x over previous
import jax
import jax.numpy as jnp
from jax import lax
from jax.experimental import pallas as pl
from jax.experimental.pallas import tpu as pltpu
from jax.experimental.pallas import tpu_sc as plsc

B = 16384
D = 64
L = 16
NC = 2
NS = 16
NW = NC * NS
ROWS = B // NW
CROWS = B // NC
HROWS = CROWS // 2
CH = 128
NCH = ROWS // CH
NG = ROWS // L
WB = D + L


def _sc_body(x_hbm, z_hbm, idx_hbm, wb_hbm, emb1_hbm, emb2_hbm,
             out_hbm, idx_v, a_v, b_v, x_v, z_v, wb_v, out_v,
             sh, sem, sem_sh):
    cid = lax.axis_index("c")
    sid = lax.axis_index("s")
    wid = cid * NS + sid
    base = wid * ROWS
    cbase = cid * CROWS

    copies = []

    def fire(src, dst):
        c = pltpu.make_async_copy(src, dst, sem)
        c.start()
        copies.append(c)

    pltpu.sync_copy(idx_hbm.at[pl.ds(wid * NCH, NCH)], idx_v)
    for j in range(NCH):
        fire(emb1_hbm.at[idx_v.at[j]], a_v.at[pl.ds(j * CH, CH)])
        fire(emb2_hbm.at[idx_v.at[j]], b_v.at[pl.ds(j * CH, CH)])
    fire(wb_hbm, wb_v)

    for arr_hbm, dst_v in ((z_hbm, z_v), (x_hbm, x_v)):
        for h in range(2):
            @pl.when(sid == 0)
            def _dma(arr_hbm=arr_hbm, h=h):
                c = pltpu.make_async_copy(
                    arr_hbm.at[pl.ds(cbase + h * HROWS, HROWS)], sh, sem_sh)
                c.start()
                c.wait()

            plsc.subcore_barrier()

            @pl.when(sid // 8 == h)
            def _pull(dst_v=dst_v, h=h):
                pltpu.sync_copy(
                    sh.at[pl.ds((sid % 8) * ROWS, ROWS)], dst_v)

            plsc.subcore_barrier()

    for c in copies:
        c.wait()

    wregs = [wb_v[0, pl.ds(k * L, L)] for k in range(D // L)]
    bias_vec = wb_v[0, pl.ds(D, L)]
    lanes = lax.iota(jnp.int32, L)

    def group(g, carry):
        out16 = jnp.zeros((L,), jnp.float32)
        for rr in range(L):
            r = g * L + rr
            acc = z_v[r, pl.ds(0, L)] * a_v[r, pl.ds(0, L)]
            for k in range(1, D // L):
                acc = acc + z_v[r, pl.ds(k * L, L)] * a_v[r, pl.ds(k * L, L)]
            for k in range(D // L):
                acc = acc + x_v[r, pl.ds(k * L, L)] * wregs[k]
            s = jnp.sum(acc)
            out16 = jnp.where(lanes == rr, s, out16)
        bv = b_v[pl.dslice(g * L, L)]
        out_v[pl.dslice(g * L, L)] = out16 + bv + bias_vec
        return carry

    lax.fori_loop(0, NG, group, 0)

    pltpu.sync_copy(out_v, out_hbm.at[pl.ds(base, ROWS)])


def _build_sc():
    mesh = plsc.VectorSubcoreMesh(core_axis_name="c", subcore_axis_name="s")
    return pl.kernel(
        _sc_body,
        out_type=jax.ShapeDtypeStruct((B,), jnp.float32),
        mesh=mesh,
        compiler_params=pltpu.CompilerParams(
            needs_layout_passes=False, use_tc_tiling_on_sc=False),
        scratch_types=[
            pltpu.VMEM((NCH, CH), jnp.int32),
            pltpu.VMEM((ROWS, D), jnp.float32),
            pltpu.VMEM((ROWS,), jnp.float32),
            pltpu.VMEM((ROWS, D), jnp.float32),
            pltpu.VMEM((ROWS, D), jnp.float32),
            pltpu.VMEM((1, WB), jnp.float32),
            pltpu.VMEM((ROWS,), jnp.float32),
            pltpu.VMEM_SHARED((HROWS, D), jnp.float32),
            pltpu.SemaphoreType.DMA,
            pltpu.SemaphoreType.DMA,
        ],
    )


_sc_kernel = _build_sc()


@jax.jit
def kernel(x, z, idx, W_f, b_f, emb1, emb2):
    wb = jnp.concatenate([W_f, jnp.broadcast_to(b_f, (1, L))], axis=1)
    idx2 = idx.astype(jnp.int32).reshape(NW * NCH, CH)
    out = _sc_kernel(x, z, idx2, wb, emb1, emb2.reshape(-1))
    return out.reshape(B, 1)
